# Initial kernel scaffold; baseline (speedup 1.0000x reference)
import jax
import jax.numpy as jnp
from jax.experimental import pallas as pl


def kernel(x):
    raise NotImplementedError("write your pallas kernel here")


# speedup vs baseline: 2.9472x; 2.9472x over previous
import functools

import jax
import jax.numpy as jnp
from jax import lax
from jax.experimental import pallas as pl
from jax.experimental.pallas import tpu as pltpu
from jax.experimental.pallas import tpu_sc as plsc

ROWS = 128
N = 32768
HALF = N // 2
NVREG = N // 16
NBINS = 2048
SIGN = jnp.int32(-2147483648)


def _keys(v_f32):
    b = plsc.bitcast(v_f32, jnp.int32)
    m = lax.shift_right_arithmetic(b, 31)
    return lax.bitwise_xor(b, lax.bitwise_or(m, SIGN))


def _zero_hist(hist):
    zeros = jnp.zeros((16,), jnp.int32)

    def body(i, _):
        hist[pl.ds(i * 16, 16)] = zeros
        return 0

    lax.fori_loop(0, NBINS // 16, body, 0)


def _excl_scan(hist):
    def body(i, carry):
        h = hist[pl.ds(i * 16, 16)]
        c = plsc.cumsum(h)
        hist[pl.ds(i * 16, 16)] = c - h + carry
        return carry + jnp.sum(h)

    lax.fori_loop(0, NBINS // 16, body, jnp.int32(0))


def _radix_pass(src_digit_payload, dst, hist):
    _zero_hist(hist)
    ones = jnp.ones((16,), jnp.int32)

    def hist_body(i, _):
        d, _p = src_digit_payload(i)
        plsc.addupdate_scatter(hist, [d], ones)
        return 0

    lax.fori_loop(0, NVREG, hist_body, 0)
    _excl_scan(hist)

    def perm_body(i, _):
        d, p = src_digit_payload(i)
        cnt, last = plsc.scan_count(d)
        base = plsc.load_gather(hist, [d])
        plsc.store_scatter(dst, [base + cnt - 1], p)
        plsc.addupdate_scatter(hist, [d], cnt, mask=last)
        return 0

    lax.fori_loop(0, NVREG, perm_body, 0)


def _body(x_hbm, out_hbm, a, b, c, hist):
    cid = lax.axis_index("c")
    sid = lax.axis_index("s")
    wid = sid * 2 + cid

    def do_row(rr, _):
        row = wid * 4 + rr
        pltpu.sync_copy(x_hbm.at[row], a)

        def src1(i):
            key = _keys(a[pl.ds(i * 16, 16)])
            d = lax.bitwise_and(key, jnp.int32(0x7FF))
            bit = jnp.where(i < (HALF // 16), jnp.int32(1), jnp.int32(0))
            p = lax.bitwise_or(
                lax.shift_left(lax.shift_right_logical(key, 11), 1), bit
            )
            return d, p

        _radix_pass(src1, b, hist)

        def src2(i):
            w = b[pl.ds(i * 16, 16)]
            d = lax.bitwise_and(lax.shift_right_logical(w, 1), jnp.int32(0x7FF))
            p = lax.bitwise_or(
                lax.shift_left(lax.shift_right_logical(w, 12), 1),
                lax.bitwise_and(w, jnp.int32(1)),
            )
            return d, p

        _radix_pass(src2, c, hist)

        def src3(i):
            w = c[pl.ds(i * 16, 16)]
            d = lax.shift_right_logical(w, 1)
            p = lax.bitwise_and(w, jnp.int32(1))
            return d, p

        _radix_pass(src3, b, hist)

        pltpu.sync_copy(b, out_hbm.at[row])
        return 0

    lax.fori_loop(0, ROWS // 32, do_row, 0)


@jax.jit
def _feature_select(x):
    mesh = plsc.VectorSubcoreMesh(core_axis_name="c", subcore_axis_name="s")
    run = functools.partial(
        pl.kernel,
        out_type=jax.ShapeDtypeStruct((ROWS, N), jnp.int32),
        mesh=mesh,
        scratch_types=[
            pltpu.VMEM((N,), jnp.float32),
            pltpu.VMEM((N,), jnp.int32),
            pltpu.VMEM((N,), jnp.int32),
            pltpu.VMEM((NBINS,), jnp.int32),
        ],
        compiler_params=pltpu.CompilerParams(needs_layout_passes=False),
    )(_body)
    return run(x)


def kernel(x):
    return _feature_select(x)


# speedup vs baseline: 3.5365x; 1.2000x over previous
import functools

import jax
import jax.numpy as jnp
from jax import lax
from jax.experimental import pallas as pl
from jax.experimental.pallas import tpu as pltpu
from jax.experimental.pallas import tpu_sc as plsc

ROWS = 128
N = 32768
HALF = N // 2
NVREG = N // 16
NBINS = 2048
SIGN = jnp.int32(-2147483648)
ONE = jnp.int32(1)


def _keys(v_f32):
    b = plsc.bitcast(v_f32, jnp.int32)
    m = lax.shift_right_arithmetic(b, 31)
    return lax.bitwise_xor(b, lax.bitwise_or(m, SIGN))


def _excl_scan_and_zero(hist, other):
    zeros = jnp.zeros((16,), jnp.int32)

    def body(i, carry):
        h = hist[pl.ds(i * 16, 16)]
        c = plsc.cumsum(h)
        hist[pl.ds(i * 16, 16)] = c - h + carry
        other[pl.ds(i * 16, 16)] = zeros
        return carry + jnp.sum(h)

    lax.fori_loop(0, NBINS // 16, body, jnp.int32(0))


def _perm_sweep(src_digit_payload, dst, offs, next_hist, next_digit):
    def body(i, _):
        d, p = src_digit_payload(i)
        cnt, last = plsc.scan_count(d)
        base = plsc.load_gather(offs, [d])
        plsc.store_scatter(dst, [base + cnt - 1], p)
        plsc.addupdate_scatter(offs, [d], cnt, mask=last)
        if next_hist is not None:
            ones = jnp.full((16,), 1, jnp.int32)
            plsc.addupdate_scatter(next_hist, [next_digit(p)], ones)
        return 0

    lax.fori_loop(0, NVREG, body, 0)


def _d2(w):
    return lax.bitwise_and(lax.shift_right_logical(w, 1), jnp.int32(0x7FF))


def _d3(w):
    return lax.shift_right_logical(w, 1)


def _body(x_hbm, out_hbm, a, b, c, hist1, hist2):
    cid = lax.axis_index("c")
    sid = lax.axis_index("s")
    wid = sid * 2 + cid

    def do_row(rr, _):
        row = wid * 4 + rr
        pltpu.sync_copy(x_hbm.at[row], a)

        zeros = jnp.zeros((16,), jnp.int32)

        def z_body(i, _):
            hist1[pl.ds(i * 16, 16)] = zeros
            return 0

        lax.fori_loop(0, NBINS // 16, z_body, 0)

        def src1(i):
            key = _keys(a[pl.ds(i * 16, 16)])
            d = lax.bitwise_and(key, jnp.int32(0x7FF))
            bit = jnp.where(i < (HALF // 16), ONE, jnp.int32(0))
            p = lax.bitwise_or(
                lax.shift_left(lax.shift_right_logical(key, 11), 1), bit
            )
            return d, p

        ones = jnp.full((16,), 1, jnp.int32)

        def h1_body(i, _):
            d, _p = src1(i)
            plsc.addupdate_scatter(hist1, [d], ones)
            return 0

        lax.fori_loop(0, NVREG, h1_body, 0)
        _excl_scan_and_zero(hist1, hist2)

        _perm_sweep(src1, b, hist1, hist2, _d2)
        _excl_scan_and_zero(hist2, hist1)

        def src2(i):
            w = b[pl.ds(i * 16, 16)]
            d = _d2(w)
            p = lax.bitwise_or(
                lax.shift_left(lax.shift_right_logical(w, 12), 1),
                lax.bitwise_and(w, ONE),
            )
            return d, p

        _perm_sweep(src2, c, hist2, hist1, _d3)
        _excl_scan_and_zero(hist1, hist2)

        def src3(i):
            w = c[pl.ds(i * 16, 16)]
            return _d3(w), lax.bitwise_and(w, ONE)

        _perm_sweep(src3, b, hist1, None, None)

        pltpu.sync_copy(b, out_hbm.at[row])
        return 0

    lax.fori_loop(0, ROWS // 32, do_row, 0)


@jax.jit
def _feature_select(x):
    mesh = plsc.VectorSubcoreMesh(core_axis_name="c", subcore_axis_name="s")
    run = functools.partial(
        pl.kernel,
        out_type=jax.ShapeDtypeStruct((ROWS, N), jnp.int32),
        mesh=mesh,
        scratch_types=[
            pltpu.VMEM((N,), jnp.float32),
            pltpu.VMEM((N,), jnp.int32),
            pltpu.VMEM((N,), jnp.int32),
            pltpu.VMEM((NBINS,), jnp.int32),
            pltpu.VMEM((NBINS,), jnp.int32),
        ],
        compiler_params=pltpu.CompilerParams(needs_layout_passes=False),
    )(_body)
    return run(x)


def kernel(x):
    return _feature_select(x)


# speedup vs baseline: 3.8312x; 1.0833x over previous
import functools

import jax
import jax.numpy as jnp
from jax import lax
from jax.experimental import pallas as pl
from jax.experimental.pallas import tpu as pltpu
from jax.experimental.pallas import tpu_sc as plsc

ROWS = 128
N = 32768
HALF = N // 2
NBINS = 2048
K = 4
CHUNK_VREGS = N // 16 // K
SIGN = jnp.int32(-2147483648)
ONE = jnp.int32(1)


def _keys(v_f32):
    b = plsc.bitcast(v_f32, jnp.int32)
    m = lax.shift_right_arithmetic(b, 31)
    return lax.bitwise_xor(b, lax.bitwise_or(m, SIGN))


def _combine_scan_and_zero(hist, other):
    zeros = jnp.zeros((16,), jnp.int32)

    def body(i, carry):
        hs = [hist[pl.ds(c * NBINS + i * 16, 16)] for c in range(K)]
        total = hs[0]
        for c in range(1, K):
            total = total + hs[c]
        base = plsc.cumsum(total) - total + carry
        for c in range(K):
            hist[pl.ds(c * NBINS + i * 16, 16)] = base
            base = base + hs[c]
            other[pl.ds(c * NBINS + i * 16, 16)] = zeros
        return carry + jnp.sum(total)

    lax.fori_loop(0, NBINS // 16, body, jnp.int32(0))


def _hist_sweep(src_digit, hist):
    ones = jnp.full((16,), 1, jnp.int32)

    def body(i, _):
        for c in range(K):
            d, _p = src_digit(c, i)
            plsc.addupdate_scatter(hist, [d + c * NBINS], ones)
        return 0

    lax.fori_loop(0, CHUNK_VREGS, body, 0)


def _perm_sweep(src_digit_payload, dst, offs):
    def body(i, _):
        dps = [src_digit_payload(c, i) for c in range(K)]
        scans = [plsc.scan_count(d) for d, _p in dps]
        for c in range(K):
            d, p = dps[c]
            cnt, last = scans[c]
            dc = d + c * NBINS
            base = plsc.load_gather(offs, [dc])
            plsc.store_scatter(dst, [base + cnt - 1], p)
            plsc.addupdate_scatter(offs, [dc], cnt, mask=last)
        return 0

    lax.fori_loop(0, CHUNK_VREGS, body, 0)


def _body(x_hbm, out_hbm, a, b, c_buf, hist1, hist2):
    cid = lax.axis_index("c")
    sid = lax.axis_index("s")
    wid = sid * 2 + cid

    def do_row(rr, _):
        row = wid * 4 + rr
        pltpu.sync_copy(x_hbm.at[row], a)

        zeros = jnp.zeros((16,), jnp.int32)

        def z_body(i, _):
            hist1[pl.ds(i * 16, 16)] = zeros
            return 0

        lax.fori_loop(0, K * NBINS // 16, z_body, 0)

        def src1(c, i):
            key = _keys(a[pl.ds(c * (N // K) + i * 16, 16)])
            d = lax.bitwise_and(key, jnp.int32(0x7FF))
            p = lax.shift_left(lax.shift_right_logical(key, 11), 1)
            if c < K // 2:
                p = lax.bitwise_or(p, ONE)
            return d, p

        def src2(c, i):
            w = b[pl.ds(c * (N // K) + i * 16, 16)]
            d = lax.bitwise_and(lax.shift_right_logical(w, 1), jnp.int32(0x7FF))
            p = lax.bitwise_or(
                lax.shift_left(lax.shift_right_logical(w, 12), 1),
                lax.bitwise_and(w, ONE),
            )
            return d, p

        def src3(c, i):
            w = c_buf[pl.ds(c * (N // K) + i * 16, 16)]
            return lax.shift_right_logical(w, 1), lax.bitwise_and(w, ONE)

        _hist_sweep(src1, hist1)
        _combine_scan_and_zero(hist1, hist2)
        _perm_sweep(src1, b, hist1)

        _hist_sweep(src2, hist2)
        _combine_scan_and_zero(hist2, hist1)
        _perm_sweep(src2, c_buf, hist2)

        _hist_sweep(src3, hist1)
        _combine_scan_and_zero(hist1, hist2)
        _perm_sweep(src3, b, hist1)

        pltpu.sync_copy(b, out_hbm.at[row])
        return 0

    lax.fori_loop(0, ROWS // 32, do_row, 0)


@jax.jit
def _feature_select(x):
    mesh = plsc.VectorSubcoreMesh(core_axis_name="c", subcore_axis_name="s")
    run = functools.partial(
        pl.kernel,
        out_type=jax.ShapeDtypeStruct((ROWS, N), jnp.int32),
        mesh=mesh,
        scratch_types=[
            pltpu.VMEM((N,), jnp.float32),
            pltpu.VMEM((N,), jnp.int32),
            pltpu.VMEM((N,), jnp.int32),
            pltpu.VMEM((K * NBINS,), jnp.int32),
            pltpu.VMEM((K * NBINS,), jnp.int32),
        ],
        compiler_params=pltpu.CompilerParams(needs_layout_passes=False),
    )(_body)
    return run(x)


def kernel(x):
    return _feature_select(x)


# speedup vs baseline: 3.9571x; 1.0329x over previous
import functools

import jax
import jax.numpy as jnp
from jax import lax
from jax.experimental import pallas as pl
from jax.experimental.pallas import tpu as pltpu
from jax.experimental.pallas import tpu_sc as plsc

ROWS = 128
N = 32768
HALF = N // 2
NBINS = 2048
K = 4
CHUNK_VREGS = N // 16 // K
SIGN = jnp.int32(-2147483648)
ONE = jnp.int32(1)


def _keys(v_f32):
    b = plsc.bitcast(v_f32, jnp.int32)
    m = lax.shift_right_arithmetic(b, 31)
    return lax.bitwise_xor(b, lax.bitwise_or(m, SIGN))


def _combine_scan_and_zero(hists, others):
    zeros = jnp.zeros((16,), jnp.int32)

    def body(i, carry):
        sl = pl.ds(i * 16, 16)
        hs = [h[sl] for h in hists]
        total = hs[0]
        for c in range(1, K):
            total = total + hs[c]
        base = plsc.cumsum(total) - total + carry
        for c in range(K):
            hists[c][sl] = base
            base = base + hs[c]
            others[c][sl] = zeros
        return carry + jnp.sum(total)

    lax.fori_loop(0, NBINS // 16, body, jnp.int32(0))


def _hist_sweep(src_digit, hists):
    ones = jnp.full((16,), 1, jnp.int32)

    def body(i, _):
        for c in range(K):
            d, _p = src_digit(c, i)
            plsc.addupdate_scatter(hists[c], [d], ones)
        return 0

    lax.fori_loop(0, CHUNK_VREGS, body, 0)


def _perm_sweep(src_digit_payload, dst, offs):
    def body(i, _):
        dps = [src_digit_payload(c, i) for c in range(K)]
        scans = [plsc.scan_count(d) for d, _p in dps]
        for c in range(K):
            d, p = dps[c]
            cnt, last = scans[c]
            base = plsc.load_gather(offs[c], [d])
            plsc.store_scatter(dst, [base + cnt - 1], p)
            plsc.addupdate_scatter(offs[c], [d], cnt, mask=last)
        return 0

    lax.fori_loop(0, CHUNK_VREGS, body, 0)


def _body(x_hbm, out_hbm, a, b, c_buf, *hists):
    h1 = list(hists[:K])
    h2 = list(hists[K:])
    cid = lax.axis_index("c")
    sid = lax.axis_index("s")
    wid = sid * 2 + cid

    def do_row(rr, _):
        row = wid * 4 + rr
        pltpu.sync_copy(x_hbm.at[row], a)

        zeros = jnp.zeros((16,), jnp.int32)

        def z_body(i, _):
            sl = pl.ds(i * 16, 16)
            for c in range(K):
                h1[c][sl] = zeros
            return 0

        lax.fori_loop(0, NBINS // 16, z_body, 0)

        def src1(c, i):
            key = _keys(a[pl.ds(c * (N // K) + i * 16, 16)])
            d = lax.bitwise_and(key, jnp.int32(0x7FF))
            p = lax.shift_left(lax.shift_right_logical(key, 11), 1)
            if c < K // 2:
                p = lax.bitwise_or(p, ONE)
            return d, p

        def src2(c, i):
            w = b[pl.ds(c * (N // K) + i * 16, 16)]
            d = lax.bitwise_and(lax.shift_right_logical(w, 1), jnp.int32(0x7FF))
            p = lax.bitwise_or(
                lax.shift_left(lax.shift_right_logical(w, 12), 1),
                lax.bitwise_and(w, ONE),
            )
            return d, p

        def src3(c, i):
            w = c_buf[pl.ds(c * (N // K) + i * 16, 16)]
            return lax.shift_right_logical(w, 1), lax.bitwise_and(w, ONE)

        _hist_sweep(src1, h1)
        _combine_scan_and_zero(h1, h2)
        _perm_sweep(src1, b, h1)

        _hist_sweep(src2, h2)
        _combine_scan_and_zero(h2, h1)
        _perm_sweep(src2, c_buf, h2)

        _hist_sweep(src3, h1)
        _combine_scan_and_zero(h1, h2)
        _perm_sweep(src3, b, h1)

        pltpu.sync_copy(b, out_hbm.at[row])
        return 0

    lax.fori_loop(0, ROWS // 32, do_row, 0)


@jax.jit
def _feature_select(x):
    mesh = plsc.VectorSubcoreMesh(core_axis_name="c", subcore_axis_name="s")
    run = functools.partial(
        pl.kernel,
        out_type=jax.ShapeDtypeStruct((ROWS, N), jnp.int32),
        mesh=mesh,
        scratch_types=[
            pltpu.VMEM((N,), jnp.float32),
            pltpu.VMEM((N,), jnp.int32),
            pltpu.VMEM((N,), jnp.int32),
        ]
        + [pltpu.VMEM((NBINS,), jnp.int32) for _ in range(2 * K)],
        compiler_params=pltpu.CompilerParams(needs_layout_passes=False),
    )(_body)
    return run(x)


def kernel(x):
    return _feature_select(x)


# speedup vs baseline: 5.2081x; 1.3161x over previous
import functools

import jax
import jax.numpy as jnp
from jax import lax
from jax.experimental import pallas as pl
from jax.experimental.pallas import tpu as pltpu
from jax.experimental.pallas import tpu_sc as plsc

ROWS = 128
N = 32768
NBINS = 2048
K = 8
CHUNK = N // K
CHUNK_VREGS = CHUNK // 16
SIGN = -2147483648


def _keys(v_f32):
    b = plsc.bitcast(v_f32, jnp.int32)
    m = lax.shift_right_arithmetic(b, 31)
    return lax.bitwise_xor(b, lax.bitwise_or(m, jnp.int32(SIGN)))


def _scan_ga_inplace(ga):
    def body(i, carry):
        sl = pl.ds(i * 16, 16)
        hs = [g[sl] for g in ga]
        total = hs[0]
        for c in range(1, K):
            total = total + hs[c]
        base = plsc.cumsum(total) - total + carry
        for c in range(K):
            ga[c][sl] = base
            base = base + hs[c]
        return carry + jnp.sum(total)

    lax.fori_loop(0, NBINS // 16, body, jnp.int32(0))


def _transfer_scan(gb, ga):
    zeros = jnp.zeros((16,), jnp.int32)

    def body(i, carry):
        sl = pl.ds(i * 16, 16)
        hs = [gb[pl.ds(c * NBINS + i * 16, 16)] for c in range(K)]
        total = hs[0]
        for c in range(1, K):
            total = total + hs[c]
        base = plsc.cumsum(total) - total + carry
        for c in range(K):
            ga[c][sl] = base
            base = base + hs[c]
            gb[pl.ds(c * NBINS + i * 16, 16)] = zeros
        return carry + jnp.sum(total)

    lax.fori_loop(0, NBINS // 16, body, jnp.int32(0))


def _perm_sweep(src_digit_payload, dst, ga, gb, next_digit, cast_f32):
    ones = jnp.full((16,), 1, jnp.int32)

    def body(i, _):
        dps = [src_digit_payload(c, i) for c in range(K)]
        scans = [plsc.scan_count(d) for d, _p in dps]
        for c in range(K):
            d, p = dps[c]
            cnt, last = scans[c]
            base = plsc.load_gather(ga[c], [d])
            pos = base + cnt - 1
            val = plsc.bitcast(p, jnp.float32) if cast_f32 else p
            plsc.store_scatter(dst, [pos], val)
            plsc.addupdate_scatter(ga[c], [d], cnt, mask=last)
            if next_digit is not None:
                nh = lax.bitwise_or(
                    lax.shift_left(lax.shift_right_logical(pos, 12), 11),
                    next_digit(p),
                )
                plsc.addupdate_scatter(gb, [nh], ones)
        return 0

    lax.fori_loop(0, CHUNK_VREGS, body, 0)


def _d2(w):
    return lax.bitwise_and(lax.shift_right_logical(w, 1), jnp.int32(0x7FF))


def _d3(w):
    return lax.shift_right_logical(w, 1)


def _body(x_hbm, out_hbm, xb, b, gb, *ga):
    ga = list(ga)
    cid = lax.axis_index("c")
    sid = lax.axis_index("s")
    wid = sid * 2 + cid

    zeros = jnp.zeros((16,), jnp.int32)

    def zb_body(i, _):
        gb[pl.ds(i * 16, 16)] = zeros
        return 0

    lax.fori_loop(0, K * NBINS // 16, zb_body, 0)

    def do_row(rr, _):
        row = wid * 4 + rr
        pltpu.sync_copy(x_hbm.at[row], xb)

        def za_body(i, _):
            sl = pl.ds(i * 16, 16)
            for c in range(K):
                ga[c][sl] = zeros
            return 0

        lax.fori_loop(0, NBINS // 16, za_body, 0)

        def src1(c, i):
            key = _keys(xb[pl.ds(c * CHUNK + i * 16, 16)])
            d = lax.bitwise_and(key, jnp.int32(0x7FF))
            p = lax.shift_left(lax.shift_right_logical(key, 11), 1)
            if c < K // 2:
                p = lax.bitwise_or(p, jnp.int32(1))
            return d, p

        def src2(c, i):
            w = b[pl.ds(c * CHUNK + i * 16, 16)]
            p = lax.bitwise_or(
                lax.shift_left(lax.shift_right_logical(w, 12), 1),
                lax.bitwise_and(w, jnp.int32(1)),
            )
            return _d2(w), p

        def src3(c, i):
            w = plsc.bitcast(xb[pl.ds(c * CHUNK + i * 16, 16)], jnp.int32)
            return _d3(w), lax.bitwise_and(w, jnp.int32(1))

        ones = jnp.full((16,), 1, jnp.int32)

        def h1_body(i, _):
            for c in range(K):
                d, _p = src1(c, i)
                plsc.addupdate_scatter(ga[c], [d], ones)
            return 0

        lax.fori_loop(0, CHUNK_VREGS, h1_body, 0)
        _scan_ga_inplace(ga)
        _perm_sweep(src1, b, ga, gb, _d2, cast_f32=False)

        _transfer_scan(gb, ga)
        _perm_sweep(src2, xb, ga, gb, _d3, cast_f32=True)

        _transfer_scan(gb, ga)
        _perm_sweep(src3, b, ga, None, None, cast_f32=False)

        pltpu.sync_copy(b, out_hbm.at[row])
        return 0

    lax.fori_loop(0, ROWS // 32, do_row, 0)


@jax.jit
def _feature_select(x):
    mesh = plsc.VectorSubcoreMesh(core_axis_name="c", subcore_axis_name="s")
    run = functools.partial(
        pl.kernel,
        out_type=jax.ShapeDtypeStruct((ROWS, N), jnp.int32),
        mesh=mesh,
        scratch_types=[
            pltpu.VMEM((N,), jnp.float32),
            pltpu.VMEM((N,), jnp.int32),
            pltpu.VMEM((K * NBINS,), jnp.int32),
        ]
        + [pltpu.VMEM((NBINS,), jnp.int32) for _ in range(K)],
        compiler_params=pltpu.CompilerParams(needs_layout_passes=False),
    )(_body)
    return run(x)


def kernel(x):
    return _feature_select(x)


# speedup vs baseline: 11.2971x; 2.1692x over previous
import functools

import jax
import jax.numpy as jnp
from jax import lax
from jax.experimental import pallas as pl
from jax.experimental.pallas import tpu as pltpu
from jax.experimental.pallas import tpu_sc as plsc

ROWS = 128
N = 32768
NBINS = 2048
K = 8
CHUNK = N // K
CHUNK_VREGS = CHUNK // 16
SIGN = -2147483648


def _keys(v_f32):
    b = plsc.bitcast(v_f32, jnp.int32)
    m = lax.shift_right_arithmetic(b, 31)
    return lax.bitwise_xor(b, lax.bitwise_or(m, jnp.int32(SIGN)))


def _scan_ga_inplace(ga):
    def body(i, carry):
        sl = pl.ds(i * 16, 16)
        hs = [g[sl] for g in ga]
        total = hs[0]
        for c in range(1, K):
            total = total + hs[c]
        base = plsc.cumsum(total) - total + carry
        for c in range(K):
            ga[c][sl] = base
            base = base + hs[c]
        return carry + jnp.sum(total)

    lax.fori_loop(0, NBINS // 16, body, jnp.int32(0))


def _transfer_scan(gb, ga):
    zeros = jnp.zeros((16,), jnp.int32)

    def body(i, carry):
        sl = pl.ds(i * 16, 16)
        hs = [gb[pl.ds(c * NBINS + i * 16, 16)] for c in range(K)]
        total = hs[0]
        for c in range(1, K):
            total = total + hs[c]
        base = plsc.cumsum(total) - total + carry
        for c in range(K):
            ga[c][sl] = base
            base = base + hs[c]
            gb[pl.ds(c * NBINS + i * 16, 16)] = zeros
        return carry + jnp.sum(total)

    lax.fori_loop(0, NBINS // 16, body, jnp.int32(0))


def _perm_sweep(src_digit_payload, dst, ga, gb, next_digit, cast_f32):
    ones = jnp.full((16,), 1, jnp.int32)

    def body(i, _):
        dps = [src_digit_payload(c, i) for c in range(K)]
        scans = [plsc.scan_count(d) for d, _p in dps]
        bases = [plsc.load_gather(ga[c], [dps[c][0]]) for c in range(K)]
        poss = [bases[c] + scans[c][0] - 1 for c in range(K)]
        for c in range(K):
            d, p = dps[c]
            cnt, last = scans[c]
            val = plsc.bitcast(p, jnp.float32) if cast_f32 else p
            plsc.store_scatter(dst, [poss[c]], val)
            plsc.addupdate_scatter(ga[c], [d], cnt, mask=last)
        if next_digit is not None:
            for c in range(K):
                pos, p = poss[c], dps[c][1]
                nh = lax.bitwise_or(
                    lax.shift_left(lax.shift_right_logical(pos, 12), 11),
                    next_digit(p),
                )
                plsc.addupdate_scatter(gb, [nh], ones)
        return 0

    lax.fori_loop(0, CHUNK_VREGS, body, 0)


def _d2(w):
    return lax.bitwise_and(lax.shift_right_logical(w, 1), jnp.int32(0x7FF))


def _d3(w):
    return lax.shift_right_logical(w, 1)


def _body(x_hbm, out_hbm, xb, b, gb, *ga):
    ga = list(ga)
    cid = lax.axis_index("c")
    sid = lax.axis_index("s")
    wid = sid * 2 + cid

    zeros = jnp.zeros((16,), jnp.int32)

    def zb_body(i, _):
        gb[pl.ds(i * 16, 16)] = zeros
        return 0

    lax.fori_loop(0, K * NBINS // 16, zb_body, 0)

    def do_row(rr, _):
        row = wid * 4 + rr
        pltpu.sync_copy(x_hbm.at[row], xb)

        def za_body(i, _):
            sl = pl.ds(i * 16, 16)
            for c in range(K):
                ga[c][sl] = zeros
            return 0

        lax.fori_loop(0, NBINS // 16, za_body, 0)

        def src1(c, i):
            key = _keys(xb[pl.ds(c * CHUNK + i * 16, 16)])
            d = lax.bitwise_and(key, jnp.int32(0x7FF))
            p = lax.shift_left(lax.shift_right_logical(key, 11), 1)
            if c < K // 2:
                p = lax.bitwise_or(p, jnp.int32(1))
            return d, p

        def src2(c, i):
            w = b[pl.ds(c * CHUNK + i * 16, 16)]
            p = lax.bitwise_or(
                lax.shift_left(lax.shift_right_logical(w, 12), 1),
                lax.bitwise_and(w, jnp.int32(1)),
            )
            return _d2(w), p

        def src3(c, i):
            w = plsc.bitcast(xb[pl.ds(c * CHUNK + i * 16, 16)], jnp.int32)
            return _d3(w), lax.bitwise_and(w, jnp.int32(1))

        ones = jnp.full((16,), 1, jnp.int32)

        @plsc.parallel_loop(0, CHUNK_VREGS, unroll=2)
        def _h1_loop(i):
            ds = [src1(c, i)[0] for c in range(K)]
            for c in range(K):
                plsc.addupdate_scatter(ga[c], [ds[c]], ones)
        _scan_ga_inplace(ga)
        _perm_sweep(src1, b, ga, gb, _d2, cast_f32=False)

        _transfer_scan(gb, ga)
        _perm_sweep(src2, xb, ga, gb, _d3, cast_f32=True)

        _transfer_scan(gb, ga)
        _perm_sweep(src3, b, ga, None, None, cast_f32=False)

        pltpu.sync_copy(b, out_hbm.at[row])
        return 0

    lax.fori_loop(0, ROWS // 32, do_row, 0)


@jax.jit
def _feature_select(x):
    mesh = plsc.VectorSubcoreMesh(core_axis_name="c", subcore_axis_name="s")
    run = functools.partial(
        pl.kernel,
        out_type=jax.ShapeDtypeStruct((ROWS, N), jnp.int32),
        mesh=mesh,
        scratch_types=[
            pltpu.VMEM((N,), jnp.float32),
            pltpu.VMEM((N,), jnp.int32),
            pltpu.VMEM((K * NBINS,), jnp.int32),
        ]
        + [pltpu.VMEM((NBINS,), jnp.int32) for _ in range(K)],
        compiler_params=pltpu.CompilerParams(needs_layout_passes=False),
    )(_body)
    return run(x)


def kernel(x):
    return _feature_select(x)


# speedup vs baseline: 11.3652x; 1.0060x over previous
import functools

import jax
import jax.numpy as jnp
from jax import lax
from jax.experimental import pallas as pl
from jax.experimental.pallas import tpu as pltpu
from jax.experimental.pallas import tpu_sc as plsc

ROWS = 128
N = 32768
NBINS = 2048
K = 8
CHUNK = N // K
CHUNK_VREGS = CHUNK // 16
SIGN = -2147483648


def _keys(v_f32):
    b = plsc.bitcast(v_f32, jnp.int32)
    m = lax.shift_right_arithmetic(b, 31)
    return lax.bitwise_xor(b, lax.bitwise_or(m, jnp.int32(SIGN)))


def _scan_ga_inplace(ga):
    def body(i, carry):
        sl = pl.ds(i * 16, 16)
        hs = [g[sl] for g in ga]
        total = hs[0]
        for c in range(1, K):
            total = total + hs[c]
        base = plsc.cumsum(total) - total + carry
        for c in range(K):
            ga[c][sl] = base
            base = base + hs[c]
        return carry + jnp.sum(total)

    lax.fori_loop(0, NBINS // 16, body, jnp.int32(0))


def _transfer_scan(gb, ga):
    zeros = jnp.zeros((16,), jnp.int32)

    def body(i, carry):
        sl = pl.ds(i * 16, 16)
        hs = [gb[pl.ds(c * NBINS + i * 16, 16)] for c in range(K)]
        total = hs[0]
        for c in range(1, K):
            total = total + hs[c]
        base = plsc.cumsum(total) - total + carry
        for c in range(K):
            ga[c][sl] = base
            base = base + hs[c]
            gb[pl.ds(c * NBINS + i * 16, 16)] = zeros
        return carry + jnp.sum(total)

    lax.fori_loop(0, NBINS // 16, body, jnp.int32(0))


def _perm_sweep(src_digit_payload, dst, ga, gb, next_digit, cast_f32):
    ones = jnp.full((16,), 1, jnp.int32)

    def step(i):
        dps = [src_digit_payload(c, i) for c in range(K)]
        scans = [plsc.scan_count(d) for d, _p in dps]
        bases = [plsc.load_gather(ga[c], [dps[c][0]]) for c in range(K)]
        poss = [bases[c] + scans[c][0] - 1 for c in range(K)]
        for c in range(K):
            d, p = dps[c]
            cnt, last = scans[c]
            val = plsc.bitcast(p, jnp.float32) if cast_f32 else p
            plsc.store_scatter(dst, [poss[c]], val)
            plsc.addupdate_scatter(ga[c], [d], cnt, mask=last)
        if next_digit is not None:
            for c in range(K):
                pos, p = poss[c], dps[c][1]
                nh = lax.bitwise_or(
                    lax.shift_left(lax.shift_right_logical(pos, 12), 11),
                    next_digit(p),
                )
                plsc.addupdate_scatter(gb, [nh], ones)

    def body(i, _):
        step(2 * i)
        step(2 * i + 1)
        return 0

    lax.fori_loop(0, CHUNK_VREGS // 2, body, 0)


def _d2(w):
    return lax.bitwise_and(lax.shift_right_logical(w, 1), jnp.int32(0x7FF))


def _d3(w):
    return lax.shift_right_logical(w, 1)


def _body(x_hbm, out_hbm, xb, b, gb, *ga):
    ga = list(ga)
    cid = lax.axis_index("c")
    sid = lax.axis_index("s")
    wid = sid * 2 + cid

    zeros = jnp.zeros((16,), jnp.int32)

    @plsc.parallel_loop(0, K * NBINS // 16, unroll=4)
    def _zb_loop(i):
        gb[pl.ds(i * 16, 16)] = zeros

    def do_row(rr, _):
        row = wid * 4 + rr
        pltpu.sync_copy(x_hbm.at[row], xb)

        @plsc.parallel_loop(0, NBINS // 16, unroll=2)
        def _za_loop(i):
            sl = pl.ds(i * 16, 16)
            for c in range(K):
                ga[c][sl] = zeros

        def src1(c, i):
            key = _keys(xb[pl.ds(c * CHUNK + i * 16, 16)])
            d = lax.bitwise_and(key, jnp.int32(0x7FF))
            p = lax.shift_left(lax.shift_right_logical(key, 11), 1)
            if c < K // 2:
                p = lax.bitwise_or(p, jnp.int32(1))
            return d, p

        def src2(c, i):
            w = b[pl.ds(c * CHUNK + i * 16, 16)]
            p = lax.bitwise_or(
                lax.shift_left(lax.shift_right_logical(w, 12), 1),
                lax.bitwise_and(w, jnp.int32(1)),
            )
            return _d2(w), p

        def src3(c, i):
            w = plsc.bitcast(xb[pl.ds(c * CHUNK + i * 16, 16)], jnp.int32)
            return _d3(w), lax.bitwise_and(w, jnp.int32(1))

        ones = jnp.full((16,), 1, jnp.int32)

        @plsc.parallel_loop(0, CHUNK_VREGS, unroll=2)
        def _h1_loop(i):
            ds = [src1(c, i)[0] for c in range(K)]
            for c in range(K):
                plsc.addupdate_scatter(ga[c], [ds[c]], ones)
        _scan_ga_inplace(ga)
        _perm_sweep(src1, b, ga, gb, _d2, cast_f32=False)

        _transfer_scan(gb, ga)
        _perm_sweep(src2, xb, ga, gb, _d3, cast_f32=True)

        _transfer_scan(gb, ga)
        _perm_sweep(src3, b, ga, None, None, cast_f32=False)

        pltpu.sync_copy(b, out_hbm.at[row])
        return 0

    lax.fori_loop(0, ROWS // 32, do_row, 0)


@jax.jit
def _feature_select(x):
    mesh = plsc.VectorSubcoreMesh(core_axis_name="c", subcore_axis_name="s")
    run = functools.partial(
        pl.kernel,
        out_type=jax.ShapeDtypeStruct((ROWS, N), jnp.int32),
        mesh=mesh,
        scratch_types=[
            pltpu.VMEM((N,), jnp.float32),
            pltpu.VMEM((N,), jnp.int32),
            pltpu.VMEM((K * NBINS,), jnp.int32),
        ]
        + [pltpu.VMEM((NBINS,), jnp.int32) for _ in range(K)],
        compiler_params=pltpu.CompilerParams(needs_layout_passes=False),
    )(_body)
    return run(x)


def kernel(x):
    return _feature_select(x)


# speedup vs baseline: 11.4302x; 1.0057x over previous
import functools

import jax
import jax.numpy as jnp
from jax import lax
from jax.experimental import pallas as pl
from jax.experimental.pallas import tpu as pltpu
from jax.experimental.pallas import tpu_sc as plsc

ROWS = 128
N = 32768
NBINS = 2048
K = 8
CHUNK = N // K
CHUNK_VREGS = CHUNK // 16
SIGN = -2147483648


def _keys(v_f32):
    b = plsc.bitcast(v_f32, jnp.int32)
    m = lax.shift_right_arithmetic(b, 31)
    return lax.bitwise_xor(b, lax.bitwise_or(m, jnp.int32(SIGN)))


def _scan_ga_inplace(ga):
    def body(i, carry):
        sl = pl.ds(i * 16, 16)
        hs = [g[sl] for g in ga]
        total = hs[0]
        for c in range(1, K):
            total = total + hs[c]
        base = plsc.cumsum(total) - total + carry
        for c in range(K):
            ga[c][sl] = base
            base = base + hs[c]
        return carry + jnp.sum(total)

    lax.fori_loop(0, NBINS // 16, body, jnp.int32(0))


def _transfer_scan(gb, ga):
    zeros = jnp.zeros((16,), jnp.int32)

    def body(i, carry):
        sl = pl.ds(i * 16, 16)
        hs = [gb[pl.ds(c * NBINS + i * 16, 16)] for c in range(K)]
        total = hs[0]
        for c in range(1, K):
            total = total + hs[c]
        base = plsc.cumsum(total) - total + carry
        for c in range(K):
            ga[c][sl] = base
            base = base + hs[c]
            gb[pl.ds(c * NBINS + i * 16, 16)] = zeros
        return carry + jnp.sum(total)

    lax.fori_loop(0, NBINS // 16, body, jnp.int32(0))


def _perm_sweep(src_digit_payload, dst, ga, gb, next_digit, cast_f32):
    ones = jnp.full((16,), 1, jnp.int32)

    def step(i):
        dps = [src_digit_payload(c, i) for c in range(K)]
        scans = [plsc.scan_count(d) for d, _p in dps]
        bases = [plsc.load_gather(ga[c], [dps[c][0]]) for c in range(K)]
        poss = [bases[c] + scans[c][0] - 1 for c in range(K)]
        for c in range(K):
            d, p = dps[c]
            cnt, last = scans[c]
            val = plsc.bitcast(p, jnp.float32) if cast_f32 else p
            plsc.store_scatter(dst, [poss[c]], val)
            plsc.addupdate_scatter(ga[c], [d], cnt, mask=last)
        if next_digit is not None:
            for c in range(K):
                pos, p = poss[c], dps[c][1]
                nh = lax.bitwise_or(
                    lax.shift_left(lax.shift_right_logical(pos, 12), 11),
                    next_digit(p),
                )
                plsc.addupdate_scatter(gb, [nh], ones)

    def body(i, _):
        step(2 * i)
        step(2 * i + 1)
        return 0

    lax.fori_loop(0, CHUNK_VREGS // 2, body, 0)


def _d2(p):
    return lax.bitwise_and(p, jnp.int32(0x7FF))


def _d3(p):
    return lax.bitwise_and(p, jnp.int32(0x3FF))


def _body(x_hbm, out_hbm, xb, b, gb, *ga):
    ga = list(ga)
    cid = lax.axis_index("c")
    sid = lax.axis_index("s")
    wid = sid * 2 + cid

    zeros = jnp.zeros((16,), jnp.int32)

    @plsc.parallel_loop(0, K * NBINS // 16, unroll=4)
    def _zb_loop(i):
        gb[pl.ds(i * 16, 16)] = zeros

    def do_row(rr, _):
        row = wid * 4 + rr
        pltpu.sync_copy(x_hbm.at[row], xb)

        @plsc.parallel_loop(0, NBINS // 16, unroll=2)
        def _za_loop(i):
            sl = pl.ds(i * 16, 16)
            for c in range(K):
                ga[c][sl] = zeros

        def src1(c, i):
            key = _keys(xb[pl.ds(c * CHUNK + i * 16, 16)])
            d = lax.bitwise_and(key, jnp.int32(0x7FF))
            p = lax.shift_right_logical(key, 11)
            if c < K // 2:
                p = lax.bitwise_or(p, jnp.int32(SIGN))
            return d, p

        def src2(c, i):
            w = b[pl.ds(c * CHUNK + i * 16, 16)]
            d = lax.bitwise_and(w, jnp.int32(0x7FF))
            p = lax.shift_right_logical(w, 11)
            return d, p

        def src3(c, i):
            w = plsc.bitcast(xb[pl.ds(c * CHUNK + i * 16, 16)], jnp.int32)
            d = lax.bitwise_and(w, jnp.int32(0x3FF))
            return d, lax.shift_right_logical(w, 20)

        ones = jnp.full((16,), 1, jnp.int32)

        @plsc.parallel_loop(0, CHUNK_VREGS, unroll=2)
        def _h1_loop(i):
            ds = [src1(c, i)[0] for c in range(K)]
            for c in range(K):
                plsc.addupdate_scatter(ga[c], [ds[c]], ones)
        _scan_ga_inplace(ga)
        _perm_sweep(src1, b, ga, gb, _d2, cast_f32=False)

        _transfer_scan(gb, ga)
        _perm_sweep(src2, xb, ga, gb, _d3, cast_f32=True)

        _transfer_scan(gb, ga)
        _perm_sweep(src3, b, ga, None, None, cast_f32=False)

        pltpu.sync_copy(b, out_hbm.at[row])
        return 0

    lax.fori_loop(0, ROWS // 32, do_row, 0)


@jax.jit
def _feature_select(x):
    mesh = plsc.VectorSubcoreMesh(core_axis_name="c", subcore_axis_name="s")
    run = functools.partial(
        pl.kernel,
        out_type=jax.ShapeDtypeStruct((ROWS, N), jnp.int32),
        mesh=mesh,
        scratch_types=[
            pltpu.VMEM((N,), jnp.float32),
            pltpu.VMEM((N,), jnp.int32),
            pltpu.VMEM((K * NBINS,), jnp.int32),
        ]
        + [pltpu.VMEM((NBINS,), jnp.int32) for _ in range(K)],
        compiler_params=pltpu.CompilerParams(needs_layout_passes=False),
    )(_body)
    return run(x)


def kernel(x):
    return _feature_select(x)


# speedup vs baseline: 11.9983x; 1.0497x over previous
import functools

import jax
import jax.numpy as jnp
from jax import lax
from jax.experimental import pallas as pl
from jax.experimental.pallas import tpu as pltpu
from jax.experimental.pallas import tpu_sc as plsc

ROWS = 128
N = 32768
NBINS = 2048
K = 8
CHUNK = N // K
CHUNK_VREGS = CHUNK // 16
SIGN = -2147483648


def _keys(v_f32):
    b = plsc.bitcast(v_f32, jnp.int32)
    m = lax.shift_right_arithmetic(b, 31)
    return lax.bitwise_xor(b, lax.bitwise_or(m, jnp.int32(SIGN)))


def _scan_ga_inplace(ga):
    def body(i, carry):
        sl = pl.ds(i * 16, 16)
        hs = [g[sl] for g in ga]
        total = hs[0]
        for c in range(1, K):
            total = total + hs[c]
        base = plsc.cumsum(total) - total + carry
        for c in range(K):
            ga[c][sl] = base
            base = base + hs[c]
        return carry + jnp.sum(total)

    lax.fori_loop(0, NBINS // 16, body, jnp.int32(0))


def _transfer_scan(gb, ga):
    zeros = jnp.zeros((16,), jnp.int32)

    def body(i, carry):
        sl = pl.ds(i * 16, 16)
        hs = [gb[pl.ds(c * NBINS + i * 16, 16)] for c in range(K)]
        total = hs[0]
        for c in range(1, K):
            total = total + hs[c]
        base = plsc.cumsum(total) - total + carry
        for c in range(K):
            ga[c][sl] = base
            base = base + hs[c]
            gb[pl.ds(c * NBINS + i * 16, 16)] = zeros
        return carry + jnp.sum(total)

    lax.fori_loop(0, NBINS // 16, body, jnp.int32(0))


def _perm_sweep(src_digit_payload, dst, ga, gb, next_digit, cast_f32):
    ones = jnp.full((16,), 1, jnp.int32)

    def load_dp(i):
        dps = [src_digit_payload(c, i) for c in range(K)]
        return tuple(d for d, _p in dps) + tuple(p for _d, p in dps)

    def process(carry):
        ds, ps = carry[:K], carry[K:]
        scans = [plsc.scan_count(d) for d in ds]
        bases = [plsc.load_gather(ga[c], [ds[c]]) for c in range(K)]
        poss = [bases[c] + scans[c][0] - 1 for c in range(K)]
        for c in range(K):
            cnt, last = scans[c]
            val = plsc.bitcast(ps[c], jnp.float32) if cast_f32 else ps[c]
            plsc.store_scatter(dst, [poss[c]], val)
            plsc.addupdate_scatter(ga[c], [ds[c]], cnt, mask=last)
        if next_digit is not None:
            for c in range(K):
                nh = lax.bitwise_or(
                    lax.shift_left(lax.shift_right_logical(poss[c], 12), 11),
                    next_digit(ps[c]),
                )
                plsc.addupdate_scatter(gb, [nh], ones)

    def body(i, carry):
        nxt = load_dp(i + 1)
        process(carry)
        return nxt

    last_carry = lax.fori_loop(0, CHUNK_VREGS - 1, body, load_dp(0))
    process(last_carry)


def _d2(p):
    return lax.bitwise_and(p, jnp.int32(0x7FF))


def _d3(p):
    return lax.bitwise_and(p, jnp.int32(0x3FF))


def _body(x_hbm, out_hbm, xb, b, gb, *ga):
    ga = list(ga)
    cid = lax.axis_index("c")
    sid = lax.axis_index("s")
    wid = sid * 2 + cid

    zeros = jnp.zeros((16,), jnp.int32)

    @plsc.parallel_loop(0, K * NBINS // 16, unroll=4)
    def _zb_loop(i):
        gb[pl.ds(i * 16, 16)] = zeros

    def do_row(rr, _):
        row = wid * 4 + rr
        pltpu.sync_copy(x_hbm.at[row], xb)

        @plsc.parallel_loop(0, NBINS // 16, unroll=2)
        def _za_loop(i):
            sl = pl.ds(i * 16, 16)
            for c in range(K):
                ga[c][sl] = zeros

        def src1(c, i):
            key = _keys(xb[pl.ds(c * CHUNK + i * 16, 16)])
            d = lax.bitwise_and(key, jnp.int32(0x7FF))
            p = lax.shift_right_logical(key, 11)
            if c < K // 2:
                p = lax.bitwise_or(p, jnp.int32(SIGN))
            return d, p

        def src2(c, i):
            w = b[pl.ds(c * CHUNK + i * 16, 16)]
            d = lax.bitwise_and(w, jnp.int32(0x7FF))
            p = lax.shift_right_logical(w, 11)
            return d, p

        def src3(c, i):
            w = plsc.bitcast(xb[pl.ds(c * CHUNK + i * 16, 16)], jnp.int32)
            d = lax.bitwise_and(w, jnp.int32(0x3FF))
            return d, lax.shift_right_logical(w, 20)

        ones = jnp.full((16,), 1, jnp.int32)

        @plsc.parallel_loop(0, CHUNK_VREGS, unroll=2)
        def _h1_loop(i):
            ds = [src1(c, i)[0] for c in range(K)]
            for c in range(K):
                plsc.addupdate_scatter(ga[c], [ds[c]], ones)
        _scan_ga_inplace(ga)
        _perm_sweep(src1, b, ga, gb, _d2, cast_f32=False)

        _transfer_scan(gb, ga)
        _perm_sweep(src2, xb, ga, gb, _d3, cast_f32=True)

        _transfer_scan(gb, ga)
        _perm_sweep(src3, b, ga, None, None, cast_f32=False)

        pltpu.sync_copy(b, out_hbm.at[row])
        return 0

    lax.fori_loop(0, ROWS // 32, do_row, 0)


@jax.jit
def _feature_select(x):
    mesh = plsc.VectorSubcoreMesh(core_axis_name="c", subcore_axis_name="s")
    run = functools.partial(
        pl.kernel,
        out_type=jax.ShapeDtypeStruct((ROWS, N), jnp.int32),
        mesh=mesh,
        scratch_types=[
            pltpu.VMEM((N,), jnp.float32),
            pltpu.VMEM((N,), jnp.int32),
            pltpu.VMEM((K * NBINS,), jnp.int32),
        ]
        + [pltpu.VMEM((NBINS,), jnp.int32) for _ in range(K)],
        compiler_params=pltpu.CompilerParams(needs_layout_passes=False),
    )(_body)
    return run(x)


def kernel(x):
    return _feature_select(x)


# speedup vs baseline: 12.3103x; 1.0260x over previous
import functools

import jax
import jax.numpy as jnp
from jax import lax
from jax.experimental import pallas as pl
from jax.experimental.pallas import tpu as pltpu
from jax.experimental.pallas import tpu_sc as plsc

ROWS = 128
N = 32768
NBINS = 2048
K = 8
CHUNK = N // K
CHUNK_VREGS = CHUNK // 16
SIGN = -2147483648


def _keys(v_f32):
    b = plsc.bitcast(v_f32, jnp.int32)
    m = lax.shift_right_arithmetic(b, 31)
    return lax.bitwise_xor(b, lax.bitwise_or(m, jnp.int32(SIGN)))


def _chunk_prefixes(hs):
    t01 = hs[0] + hs[1]
    t23 = hs[2] + hs[3]
    t45 = hs[4] + hs[5]
    t67 = hs[6] + hs[7]
    pre4 = t01 + t23
    t4567 = t45 + t67
    pre = [
        None,
        hs[0],
        t01,
        t01 + hs[2],
        pre4,
        pre4 + hs[4],
        pre4 + t45,
        pre4 + t45 + hs[6],
    ]
    return pre, pre4 + t4567


def _scan_step(load_h, store_off, carry, zero_src):
    hs = [load_h(c) for c in range(K)]
    pre, total = _chunk_prefixes(hs)
    incl = plsc.cumsum(total)
    base = incl - total + carry
    store_off(0, base)
    for c in range(1, K):
        store_off(c, base + pre[c])
    if zero_src is not None:
        zeros = jnp.zeros((16,), jnp.int32)
        for c in range(K):
            zero_src(c, zeros)
    return carry + jnp.sum(total)


def _scan_ga_inplace(ga):
    def body(i, carry):
        for j in (2 * i, 2 * i + 1):
            sl = pl.ds(j * 16, 16)
            carry = _scan_step(
                lambda c: ga[c][sl],
                lambda c, v: ga[c].__setitem__(sl, v),
                carry,
                None,
            )
        return carry

    lax.fori_loop(0, NBINS // 32, body, jnp.int32(0))


def _transfer_scan(gb, ga):
    def body(i, carry):
        for j in (2 * i, 2 * i + 1):
            sl = pl.ds(j * 16, 16)

            def gbsl(c, j=j):
                return pl.ds(c * NBINS + j * 16, 16)

            carry = _scan_step(
                lambda c: gb[gbsl(c)],
                lambda c, v: ga[c].__setitem__(sl, v),
                carry,
                lambda c, z: gb.__setitem__(gbsl(c), z),
            )
        return carry

    lax.fori_loop(0, NBINS // 32, body, jnp.int32(0))


def _perm_sweep(src_digit_payload, dst, ga, gb, next_digit, cast_f32):
    ones = jnp.full((16,), 1, jnp.int32)

    def load_dp(i):
        dps = [src_digit_payload(c, i) for c in range(K)]
        return tuple(d for d, _p in dps) + tuple(p for _d, p in dps)

    def process(carry):
        ds, ps = carry[:K], carry[K:]
        scans = [plsc.scan_count(d) for d in ds]
        bases = [plsc.load_gather(ga[c], [ds[c]]) for c in range(K)]
        poss = [bases[c] + scans[c][0] - 1 for c in range(K)]
        for c in range(K):
            cnt, last = scans[c]
            val = plsc.bitcast(ps[c], jnp.float32) if cast_f32 else ps[c]
            plsc.store_scatter(dst, [poss[c]], val)
            plsc.addupdate_scatter(ga[c], [ds[c]], cnt, mask=last)
        if next_digit is not None:
            for c in range(K):
                nh = lax.bitwise_or(
                    lax.shift_left(lax.shift_right_logical(poss[c], 12), 11),
                    next_digit(ps[c]),
                )
                plsc.addupdate_scatter(gb, [nh], ones)

    def body(i, carry):
        nxt = load_dp(i + 1)
        process(carry)
        return nxt

    last_carry = lax.fori_loop(0, CHUNK_VREGS - 1, body, load_dp(0))
    process(last_carry)


def _d2(p):
    return lax.bitwise_and(p, jnp.int32(0x7FF))


def _d3(p):
    return lax.bitwise_and(p, jnp.int32(0x3FF))


def _body(x_hbm, out_hbm, xb, b, gb, *ga):
    ga = list(ga)
    cid = lax.axis_index("c")
    sid = lax.axis_index("s")
    wid = sid * 2 + cid

    zeros = jnp.zeros((16,), jnp.int32)

    @plsc.parallel_loop(0, K * NBINS // 16, unroll=4)
    def _zb_loop(i):
        gb[pl.ds(i * 16, 16)] = zeros

    def do_row(rr, _):
        row = wid * 4 + rr
        pltpu.sync_copy(x_hbm.at[row], xb)

        @plsc.parallel_loop(0, NBINS // 16, unroll=2)
        def _za_loop(i):
            sl = pl.ds(i * 16, 16)
            for c in range(K):
                ga[c][sl] = zeros

        def src1(c, i):
            key = _keys(xb[pl.ds(c * CHUNK + i * 16, 16)])
            d = lax.bitwise_and(key, jnp.int32(0x7FF))
            p = lax.shift_right_logical(key, 11)
            if c < K // 2:
                p = lax.bitwise_or(p, jnp.int32(SIGN))
            return d, p

        def src2(c, i):
            w = b[pl.ds(c * CHUNK + i * 16, 16)]
            d = lax.bitwise_and(w, jnp.int32(0x7FF))
            p = lax.shift_right_logical(w, 11)
            return d, p

        def src3(c, i):
            w = plsc.bitcast(xb[pl.ds(c * CHUNK + i * 16, 16)], jnp.int32)
            d = lax.bitwise_and(w, jnp.int32(0x3FF))
            return d, lax.shift_right_logical(w, 20)

        ones = jnp.full((16,), 1, jnp.int32)

        @plsc.parallel_loop(0, CHUNK_VREGS, unroll=2)
        def _h1_loop(i):
            ds = [src1(c, i)[0] for c in range(K)]
            for c in range(K):
                plsc.addupdate_scatter(ga[c], [ds[c]], ones)
        _scan_ga_inplace(ga)
        _perm_sweep(src1, b, ga, gb, _d2, cast_f32=False)

        _transfer_scan(gb, ga)
        _perm_sweep(src2, xb, ga, gb, _d3, cast_f32=True)

        _transfer_scan(gb, ga)
        _perm_sweep(src3, b, ga, None, None, cast_f32=False)

        pltpu.sync_copy(b, out_hbm.at[row])
        return 0

    lax.fori_loop(0, ROWS // 32, do_row, 0)


@jax.jit
def _feature_select(x):
    mesh = plsc.VectorSubcoreMesh(core_axis_name="c", subcore_axis_name="s")
    run = functools.partial(
        pl.kernel,
        out_type=jax.ShapeDtypeStruct((ROWS, N), jnp.int32),
        mesh=mesh,
        scratch_types=[
            pltpu.VMEM((N,), jnp.float32),
            pltpu.VMEM((N,), jnp.int32),
            pltpu.VMEM((K * NBINS,), jnp.int32),
        ]
        + [pltpu.VMEM((NBINS,), jnp.int32) for _ in range(K)],
        compiler_params=pltpu.CompilerParams(needs_layout_passes=False),
    )(_body)
    return run(x)


def kernel(x):
    return _feature_select(x)


# speedup vs baseline: 12.3438x; 1.0027x over previous
import functools

import jax
import jax.numpy as jnp
from jax import lax
from jax.experimental import pallas as pl
from jax.experimental.pallas import tpu as pltpu
from jax.experimental.pallas import tpu_sc as plsc

ROWS = 128
N = 32768
NBINS = 2048
K = 8
CHUNK = N // K
CHUNK_VREGS = CHUNK // 16
SIGN = -2147483648


def _keys(v_f32):
    b = plsc.bitcast(v_f32, jnp.int32)
    m = lax.shift_right_arithmetic(b, 31)
    return lax.bitwise_xor(b, lax.bitwise_or(m, jnp.int32(SIGN)))


def _chunk_prefixes(hs):
    t01 = hs[0] + hs[1]
    t23 = hs[2] + hs[3]
    t45 = hs[4] + hs[5]
    t67 = hs[6] + hs[7]
    pre4 = t01 + t23
    t4567 = t45 + t67
    pre = [
        None,
        hs[0],
        t01,
        t01 + hs[2],
        pre4,
        pre4 + hs[4],
        pre4 + t45,
        pre4 + t45 + hs[6],
    ]
    return pre, pre4 + t4567


def _scan_step(load_h, store_off, carry, zero_src):
    hs = [load_h(c) for c in range(K)]
    pre, total = _chunk_prefixes(hs)
    incl = plsc.cumsum(total)
    base = incl - total + carry
    store_off(0, base)
    for c in range(1, K):
        store_off(c, base + pre[c])
    if zero_src is not None:
        zeros = jnp.zeros((16,), jnp.int32)
        for c in range(K):
            zero_src(c, zeros)
    return carry + jnp.sum(total)


def _scan_ga_inplace(ga):
    def body(i, carry):
        for j in (2 * i, 2 * i + 1):
            sl = pl.ds(j * 16, 16)
            carry = _scan_step(
                lambda c: ga[c][sl],
                lambda c, v: ga[c].__setitem__(sl, v),
                carry,
                None,
            )
        return carry

    lax.fori_loop(0, NBINS // 32, body, jnp.int32(0))


def _transfer_scan(gb, ga):
    def body(i, carry):
        for j in (2 * i, 2 * i + 1):
            sl = pl.ds(j * 16, 16)

            def gbsl(c, j=j):
                return pl.ds(c * NBINS + j * 16, 16)

            carry = _scan_step(
                lambda c: gb[gbsl(c)],
                lambda c, v: ga[c].__setitem__(sl, v),
                carry,
                lambda c, z: gb.__setitem__(gbsl(c), z),
            )
        return carry

    lax.fori_loop(0, NBINS // 32, body, jnp.int32(0))


def _perm_sweep(src_digit_payload, dst, ga, gb, next_digit, cast_f32):
    ones = jnp.full((16,), 1, jnp.int32)

    def load_dp(i):
        dps = [src_digit_payload(c, i) for c in range(K)]
        return tuple(d for d, _p in dps) + tuple(p for _d, p in dps)

    def process(carry):
        ds, ps = carry[:K], carry[K:]
        scans = [plsc.scan_count(d) for d in ds]
        bases = [plsc.load_gather(ga[c], [ds[c]]) for c in range(K)]
        poss = [bases[c] + scans[c][0] - 1 for c in range(K)]
        for c in range(K):
            cnt, last = scans[c]
            val = plsc.bitcast(ps[c], jnp.float32) if cast_f32 else ps[c]
            plsc.store_scatter(dst, [poss[c]], val)
            plsc.addupdate_scatter(ga[c], [ds[c]], cnt, mask=last)
        if next_digit is not None:
            for c in range(K):
                nh = lax.bitwise_or(
                    lax.shift_left(lax.shift_right_logical(poss[c], 12), 11),
                    next_digit(ps[c]),
                )
                plsc.addupdate_scatter(gb, [nh], ones)

    def body(i, carry):
        nxt = load_dp(i + 1)
        process(carry)
        return nxt

    last_carry = lax.fori_loop(0, CHUNK_VREGS - 1, body, load_dp(0))
    process(last_carry)


def _d2(p):
    return lax.bitwise_and(p, jnp.int32(0x7FF))


def _d3(p):
    return lax.bitwise_and(p, jnp.int32(0x3FF))


def _body(x_hbm, out_hbm, xb, b, gb, *ga):
    ga = list(ga)
    cid = lax.axis_index("c")
    sid = lax.axis_index("s")
    wid = sid * 2 + cid

    zeros = jnp.zeros((16,), jnp.int32)

    @plsc.parallel_loop(0, K * NBINS // 16, unroll=4)
    def _zb_loop(i):
        gb[pl.ds(i * 16, 16)] = zeros

    def do_row(rr, _):
        row = wid * 4 + rr
        pltpu.sync_copy(x_hbm.at[row], xb)

        @plsc.parallel_loop(0, NBINS // 16, unroll=2)
        def _za_loop(i):
            sl = pl.ds(i * 16, 16)
            for c in range(K):
                ga[c][sl] = zeros

        def src1(c, i):
            key = _keys(xb[pl.ds(c * CHUNK + i * 16, 16)])
            d = lax.bitwise_and(key, jnp.int32(0x7FF))
            p = lax.shift_right_logical(key, 11)
            if c < K // 2:
                p = lax.bitwise_or(p, jnp.int32(SIGN))
            return d, p

        def src2(c, i):
            w = b[pl.ds(c * CHUNK + i * 16, 16)]
            d = lax.bitwise_and(w, jnp.int32(0x7FF))
            p = lax.shift_right_logical(w, 11)
            return d, p

        def src3(c, i):
            w = plsc.bitcast(xb[pl.ds(c * CHUNK + i * 16, 16)], jnp.int32)
            d = lax.bitwise_and(w, jnp.int32(0x3FF))
            return d, lax.shift_right_logical(w, 20)

        ones = jnp.full((16,), 1, jnp.int32)

        @plsc.parallel_loop(0, CHUNK_VREGS, unroll=4)
        def _h1_loop(i):
            ds = [src1(c, i)[0] for c in range(K)]
            for c in range(K):
                plsc.addupdate_scatter(ga[c], [ds[c]], ones)
        _scan_ga_inplace(ga)
        _perm_sweep(src1, b, ga, gb, _d2, cast_f32=False)

        _transfer_scan(gb, ga)
        _perm_sweep(src2, xb, ga, gb, _d3, cast_f32=True)

        _transfer_scan(gb, ga)
        _perm_sweep(src3, b, ga, None, None, cast_f32=False)

        pltpu.sync_copy(b, out_hbm.at[row])
        return 0

    lax.fori_loop(0, ROWS // 32, do_row, 0)


@jax.jit
def _feature_select(x):
    mesh = plsc.VectorSubcoreMesh(core_axis_name="c", subcore_axis_name="s")
    run = functools.partial(
        pl.kernel,
        out_type=jax.ShapeDtypeStruct((ROWS, N), jnp.int32),
        mesh=mesh,
        scratch_types=[
            pltpu.VMEM((N,), jnp.float32),
            pltpu.VMEM((N,), jnp.int32),
            pltpu.VMEM((K * NBINS,), jnp.int32),
        ]
        + [pltpu.VMEM((NBINS,), jnp.int32) for _ in range(K)],
        compiler_params=pltpu.CompilerParams(needs_layout_passes=False),
    )(_body)
    return run(x)


def kernel(x):
    return _feature_select(x)


# speedup vs baseline: 12.8421x; 1.0404x over previous
import functools

import jax
import jax.numpy as jnp
from jax import lax
from jax.experimental import pallas as pl
from jax.experimental.pallas import tpu as pltpu
from jax.experimental.pallas import tpu_sc as plsc

ROWS = 128
N = 32768
NBINS = 2048
K = 8
CHUNK = N // K
CHUNK_VREGS = CHUNK // 16
SIGN = -2147483648


def _keys(v_f32):
    b = plsc.bitcast(v_f32, jnp.int32)
    m = lax.shift_right_arithmetic(b, 31)
    return lax.bitwise_xor(b, lax.bitwise_or(m, jnp.int32(SIGN)))


def _chunk_prefixes(hs):
    t01 = hs[0] + hs[1]
    t23 = hs[2] + hs[3]
    t45 = hs[4] + hs[5]
    t67 = hs[6] + hs[7]
    pre4 = t01 + t23
    t4567 = t45 + t67
    pre = [
        None,
        hs[0],
        t01,
        t01 + hs[2],
        pre4,
        pre4 + hs[4],
        pre4 + t45,
        pre4 + t45 + hs[6],
    ]
    return pre, pre4 + t4567


def _scan_step(load_h, store_off, carry, zero_src):
    hs = [load_h(c) for c in range(K)]
    pre, total = _chunk_prefixes(hs)
    incl = plsc.cumsum(total)
    base = incl - total + carry
    store_off(0, base)
    for c in range(1, K):
        store_off(c, base + pre[c])
    if zero_src is not None:
        zeros = jnp.zeros((16,), jnp.int32)
        for c in range(K):
            zero_src(c, zeros)
    return carry + jnp.sum(total)


def _scan_ga_inplace(ga):
    def body(i, carry):
        for j in (4 * i, 4 * i + 1, 4 * i + 2, 4 * i + 3):
            sl = pl.ds(j * 16, 16)
            carry = _scan_step(
                lambda c: ga[c][sl],
                lambda c, v: ga[c].__setitem__(sl, v),
                carry,
                None,
            )
        return carry

    lax.fori_loop(0, NBINS // 64, body, jnp.int32(0))


def _transfer_scan(gb, ga):
    def body(i, carry):
        for j in (4 * i, 4 * i + 1, 4 * i + 2, 4 * i + 3):
            sl = pl.ds(j * 16, 16)

            def gbsl(c, j=j):
                return pl.ds(c * NBINS + j * 16, 16)

            carry = _scan_step(
                lambda c: gb[gbsl(c)],
                lambda c, v: ga[c].__setitem__(sl, v),
                carry,
                lambda c, z: gb.__setitem__(gbsl(c), z),
            )
        return carry

    lax.fori_loop(0, NBINS // 64, body, jnp.int32(0))


def _perm_sweep(src_digit_payload, dst, ga, gb, next_digit, cast_f32):
    ones = jnp.full((16,), 1, jnp.int32)

    def load_dp(i):
        dps = [src_digit_payload(c, i) for c in range(K)]
        return tuple(d for d, _p in dps) + tuple(p for _d, p in dps)

    def process(carry):
        ds, ps = carry[:K], carry[K:]
        scans = [plsc.scan_count(d) for d in ds]
        bases = [plsc.load_gather(ga[c], [ds[c]]) for c in range(K)]
        poss = [bases[c] + scans[c][0] - 1 for c in range(K)]
        for c in range(K):
            cnt, last = scans[c]
            val = plsc.bitcast(ps[c], jnp.float32) if cast_f32 else ps[c]
            plsc.store_scatter(dst, [poss[c]], val)
            plsc.addupdate_scatter(ga[c], [ds[c]], cnt, mask=last)
        if next_digit is not None:
            for c in range(K):
                nh = lax.bitwise_or(
                    lax.shift_left(lax.shift_right_logical(poss[c], 12), 11),
                    next_digit(ps[c]),
                )
                plsc.addupdate_scatter(gb, [nh], ones)

    def body(i, carry):
        nxt = load_dp(i + 1)
        process(carry)
        return nxt

    last_carry = lax.fori_loop(0, CHUNK_VREGS - 1, body, load_dp(0))
    process(last_carry)


def _d2(p):
    return lax.bitwise_and(p, jnp.int32(0x7FF))


def _d3(p):
    return lax.bitwise_and(p, jnp.int32(0x3FF))


def _body(x_hbm, out_hbm, xb, b, gb, *ga):
    ga = list(ga)
    cid = lax.axis_index("c")
    sid = lax.axis_index("s")
    wid = sid * 2 + cid

    zeros = jnp.zeros((16,), jnp.int32)

    @plsc.parallel_loop(0, K * NBINS // 16, unroll=4)
    def _zb_loop(i):
        gb[pl.ds(i * 16, 16)] = zeros

    def do_row(rr, _):
        row = wid * 4 + rr
        pltpu.sync_copy(x_hbm.at[row], xb)

        @plsc.parallel_loop(0, NBINS // 16, unroll=2)
        def _za_loop(i):
            sl = pl.ds(i * 16, 16)
            for c in range(K):
                ga[c][sl] = zeros

        def src1(c, i):
            key = _keys(xb[pl.ds(c * CHUNK + i * 16, 16)])
            d = lax.bitwise_and(key, jnp.int32(0x7FF))
            p = lax.shift_right_logical(key, 11)
            if c < K // 2:
                p = lax.bitwise_or(p, jnp.int32(SIGN))
            return d, p

        def src2(c, i):
            w = b[pl.ds(c * CHUNK + i * 16, 16)]
            d = lax.bitwise_and(w, jnp.int32(0x7FF))
            p = lax.shift_right_logical(w, 11)
            return d, p

        def src3(c, i):
            w = plsc.bitcast(xb[pl.ds(c * CHUNK + i * 16, 16)], jnp.int32)
            d = lax.bitwise_and(w, jnp.int32(0x3FF))
            return d, lax.shift_right_logical(w, 20)

        ones = jnp.full((16,), 1, jnp.int32)

        @plsc.parallel_loop(0, CHUNK_VREGS, unroll=4)
        def _h1_loop(i):
            ds = [src1(c, i)[0] for c in range(K)]
            for c in range(K):
                plsc.addupdate_scatter(ga[c], [ds[c]], ones)
        _scan_ga_inplace(ga)
        _perm_sweep(src1, b, ga, gb, _d2, cast_f32=False)

        _transfer_scan(gb, ga)
        _perm_sweep(src2, xb, ga, gb, _d3, cast_f32=True)

        _transfer_scan(gb, ga)
        _perm_sweep(src3, b, ga, None, None, cast_f32=False)

        pltpu.sync_copy(b, out_hbm.at[row])
        return 0

    lax.fori_loop(0, ROWS // 32, do_row, 0)


@jax.jit
def _feature_select(x):
    mesh = plsc.VectorSubcoreMesh(core_axis_name="c", subcore_axis_name="s")
    run = functools.partial(
        pl.kernel,
        out_type=jax.ShapeDtypeStruct((ROWS, N), jnp.int32),
        mesh=mesh,
        scratch_types=[
            pltpu.VMEM((N,), jnp.float32),
            pltpu.VMEM((N,), jnp.int32),
            pltpu.VMEM((K * NBINS,), jnp.int32),
        ]
        + [pltpu.VMEM((NBINS,), jnp.int32) for _ in range(K)],
        compiler_params=pltpu.CompilerParams(needs_layout_passes=False),
    )(_body)
    return run(x)


def kernel(x):
    return _feature_select(x)


# speedup vs baseline: 12.8934x; 1.0040x over previous
import functools

import jax
import jax.numpy as jnp
from jax import lax
from jax.experimental import pallas as pl
from jax.experimental.pallas import tpu as pltpu
from jax.experimental.pallas import tpu_sc as plsc

ROWS = 128
N = 32768
NBINS = 2048
K = 8
CHUNK = N // K
CHUNK_VREGS = CHUNK // 16
SIGN = -2147483648


def _keys(v_f32):
    b = plsc.bitcast(v_f32, jnp.int32)
    m = lax.shift_right_arithmetic(b, 31)
    return lax.bitwise_xor(b, lax.bitwise_or(m, jnp.int32(SIGN)))


def _chunk_prefixes(hs):
    t01 = hs[0] + hs[1]
    t23 = hs[2] + hs[3]
    t45 = hs[4] + hs[5]
    t67 = hs[6] + hs[7]
    pre4 = t01 + t23
    t4567 = t45 + t67
    pre = [
        None,
        hs[0],
        t01,
        t01 + hs[2],
        pre4,
        pre4 + hs[4],
        pre4 + t45,
        pre4 + t45 + hs[6],
    ]
    return pre, pre4 + t4567


def _scan_step(load_h, store_off, carry, zero_src):
    hs = [load_h(c) for c in range(K)]
    pre, total = _chunk_prefixes(hs)
    incl = plsc.cumsum(total)
    base = incl - total + carry
    store_off(0, base)
    for c in range(1, K):
        store_off(c, base + pre[c])
    if zero_src is not None:
        zeros = jnp.zeros((16,), jnp.int32)
        for c in range(K):
            zero_src(c, zeros)
    return carry + jnp.sum(total)


def _scan_ga_inplace(ga):
    def body(i, carry):
        for j in (4 * i, 4 * i + 1, 4 * i + 2, 4 * i + 3):
            sl = pl.ds(j * 16, 16)
            carry = _scan_step(
                lambda c: ga[c][sl],
                lambda c, v: ga[c].__setitem__(sl, v),
                carry,
                None,
            )
        return carry

    lax.fori_loop(0, NBINS // 64, body, jnp.int32(0))


def _transfer_scan(gb, ga):
    def body(i, carry):
        for j in (4 * i, 4 * i + 1, 4 * i + 2, 4 * i + 3):
            sl = pl.ds(j * 16, 16)

            def gbsl(c, j=j):
                return pl.ds(c * NBINS + j * 16, 16)

            carry = _scan_step(
                lambda c: gb[gbsl(c)],
                lambda c, v: ga[c].__setitem__(sl, v),
                carry,
                lambda c, z: gb.__setitem__(gbsl(c), z),
            )
        return carry

    lax.fori_loop(0, NBINS // 64, body, jnp.int32(0))


def _perm_sweep(src_digit_payload, dst, ga, gb, next_digit, cast_f32):
    ones = jnp.full((16,), 1, jnp.int32)

    def load_dp(i):
        dps = [src_digit_payload(c, i) for c in range(K)]
        return tuple(d for d, _p in dps) + tuple(p for _d, p in dps)

    def process(carry):
        ds, ps = carry[:K], carry[K:]
        scans = [plsc.scan_count(d) for d in ds]
        bases = [plsc.load_gather(ga[c], [ds[c]]) for c in range(K)]
        poss = [bases[c] + scans[c][0] - 1 for c in range(K)]
        for c in range(K):
            cnt, last = scans[c]
            val = plsc.bitcast(ps[c], jnp.float32) if cast_f32 else ps[c]
            plsc.store_scatter(dst, [poss[c]], val)
            plsc.addupdate_scatter(ga[c], [ds[c]], cnt, mask=last)
        if next_digit is not None:
            for c in range(K):
                nh = lax.bitwise_or(
                    lax.shift_left(lax.shift_right_logical(poss[c], 12), 11),
                    next_digit(ps[c]),
                )
                plsc.addupdate_scatter(gb, [nh], ones)

    def body(i, carry):
        nxt = load_dp(i + 1)
        process(carry)
        return nxt

    last_carry = lax.fori_loop(0, CHUNK_VREGS - 1, body, load_dp(0))
    process(last_carry)


def _d2(p):
    return lax.bitwise_and(p, jnp.int32(0x7FF))


def _d3(p):
    return lax.bitwise_and(p, jnp.int32(0x3FF))


def _body(x_hbm, out_hbm, xb, b, gb, *ga):
    ga = list(ga)
    cid = lax.axis_index("c")
    sid = lax.axis_index("s")
    wid = sid * 2 + cid

    zeros = jnp.zeros((16,), jnp.int32)

    @plsc.parallel_loop(0, K * NBINS // 16, unroll=8)
    def _zb_loop(i):
        gb[pl.ds(i * 16, 16)] = zeros

    def do_row(rr, _):
        row = wid * 4 + rr
        pltpu.sync_copy(x_hbm.at[row], xb)

        @plsc.parallel_loop(0, NBINS // 16, unroll=4)
        def _za_loop(i):
            sl = pl.ds(i * 16, 16)
            for c in range(K):
                ga[c][sl] = zeros

        def src1(c, i):
            key = _keys(xb[pl.ds(c * CHUNK + i * 16, 16)])
            d = lax.bitwise_and(key, jnp.int32(0x7FF))
            p = lax.shift_right_logical(key, 11)
            if c < K // 2:
                p = lax.bitwise_or(p, jnp.int32(SIGN))
            return d, p

        def src2(c, i):
            w = b[pl.ds(c * CHUNK + i * 16, 16)]
            d = lax.bitwise_and(w, jnp.int32(0x7FF))
            p = lax.shift_right_logical(w, 11)
            return d, p

        def src3(c, i):
            w = plsc.bitcast(xb[pl.ds(c * CHUNK + i * 16, 16)], jnp.int32)
            d = lax.bitwise_and(w, jnp.int32(0x3FF))
            return d, lax.shift_right_logical(w, 20)

        ones = jnp.full((16,), 1, jnp.int32)

        @plsc.parallel_loop(0, CHUNK_VREGS, unroll=4)
        def _h1_loop(i):
            ds = [src1(c, i)[0] for c in range(K)]
            for c in range(K):
                plsc.addupdate_scatter(ga[c], [ds[c]], ones)
        _scan_ga_inplace(ga)
        _perm_sweep(src1, b, ga, gb, _d2, cast_f32=False)

        _transfer_scan(gb, ga)
        _perm_sweep(src2, xb, ga, gb, _d3, cast_f32=True)

        _transfer_scan(gb, ga)
        _perm_sweep(src3, b, ga, None, None, cast_f32=False)

        pltpu.sync_copy(b, out_hbm.at[row])
        return 0

    lax.fori_loop(0, ROWS // 32, do_row, 0)


@jax.jit
def _feature_select(x):
    mesh = plsc.VectorSubcoreMesh(core_axis_name="c", subcore_axis_name="s")
    run = functools.partial(
        pl.kernel,
        out_type=jax.ShapeDtypeStruct((ROWS, N), jnp.int32),
        mesh=mesh,
        scratch_types=[
            pltpu.VMEM((N,), jnp.float32),
            pltpu.VMEM((N,), jnp.int32),
            pltpu.VMEM((K * NBINS,), jnp.int32),
        ]
        + [pltpu.VMEM((NBINS,), jnp.int32) for _ in range(K)],
        compiler_params=pltpu.CompilerParams(needs_layout_passes=False),
    )(_body)
    return run(x)


def kernel(x):
    return _feature_select(x)
